# baseline (speedup 1.0000x reference)
import functools

import numpy as np
import jax
import jax.numpy as jnp
from jax import lax
from jax.experimental import pallas as pl
from jax.experimental.pallas import tpu as pltpu
from jax.experimental.pallas import tpu_sc as plsc

_K = 3
_D = 100
_M1 = 0.05
_M2 = 0.25
_REG = 10.0

_SLAB = 128
_NW = 32
_TC_BLOCK = 8192
_TC_BLOCKS = 0


def _hinge(d):
    return jnp.maximum(_M1 - d, 0.0) + jnp.maximum(d - _M2, 0.0)


def _band_matrix():
    w = np.zeros((_K, _K * _D), np.float32)
    for p in range(_K):
        w[p, p * _D:(p + 1) * _D] = 1.0
    return w


def _tc_body(x_ref, w_ref, o_ref, *, grid, scale):
    i = pl.program_id(0)
    x = x_ref[...]
    r = jnp.roll(x, -_D, axis=0)
    z = (x - r) ** 2
    d = jax.lax.dot_general(w_ref[...], z, (((1,), (0,)), ((), ())),
                            preferred_element_type=jnp.float32)
    h = _hinge(d)
    s = jnp.sum(h)

    @pl.when(i == 0)
    def _init():
        o_ref[0, 0] = 0.0

    o_ref[0, 0] += s

    @pl.when(i == grid - 1)
    def _fin():
        o_ref[0, 0] *= scale


def kernel(user_ids, pos_ids, neg_ids, user_emb, item_emb):
    n, kd = user_emb.shape
    xt = user_emb.T
    scale = 2.0 * _REG / (n * _K * _K)
    c0 = _TC_BLOCKS * _TC_BLOCK
    slab0 = c0 // _SLAB
    nslabs = n // _SLAB
    tail = n - nslabs * _SLAB
    mesh = plsc.VectorSubcoreMesh(core_axis_name="c", subcore_axis_name="s")

    @functools.partial(
        pl.kernel,
        mesh=mesh,
        out_type=jax.ShapeDtypeStruct((_NW, 16), jnp.float32),
        scratch_types=[
            pltpu.VMEM((kd, _SLAB), jnp.float32),
            pltpu.VMEM((kd, 32), jnp.float32),
            pltpu.VMEM((16,), jnp.float32),
        ],
    )
    def sck(xt_hbm, out_hbm, buf, buft, accv):
        c = lax.axis_index("c")
        s = lax.axis_index("s")
        wid = s * 2 + c
        accv[...] = jnp.zeros((16,), jnp.float32)
        nj = (nslabs - slab0 + _NW - 1) // _NW

        def lane_groups(b, width):
            for l in range(width // 16):
                def row_body(r, carry):
                    d01, d12, d20 = carry
                    v0 = b[r, pl.ds(l * 16, 16)]
                    v1 = b[r + _D, pl.ds(l * 16, 16)]
                    v2 = b[r + 2 * _D, pl.ds(l * 16, 16)]
                    e0 = v0 - v1
                    e1 = v1 - v2
                    e2 = v2 - v0
                    return (d01 + e0 * e0, d12 + e1 * e1, d20 + e2 * e2)

                z = jnp.zeros((16,), jnp.float32)
                d01, d12, d20 = lax.fori_loop(0, _D, row_body, (z, z, z))
                accv[...] = (accv[...]
                             + _hinge(d01) + _hinge(d12) + _hinge(d20))

        def slab_body(j, _):
            sl = slab0 + wid + j * _NW

            @pl.when(sl < nslabs)
            def _do():
                pltpu.sync_copy(xt_hbm.at[:, pl.ds(sl * _SLAB, _SLAB)], buf)
                lane_groups(buf, _SLAB)
            return ()

        lax.fori_loop(0, nj, slab_body, ())

        if tail:
            @pl.when(wid == _NW - 1)
            def _tail():
                pltpu.sync_copy(xt_hbm.at[:, pl.ds(nslabs * _SLAB, tail)],
                                buft)
                lane_groups(buft, tail)

        accv[...] = accv[...] * scale
        pltpu.sync_copy(accv, out_hbm.at[wid])

    total = jnp.sum(sck(xt))

    if _TC_BLOCKS:
        wmat = jnp.asarray(_band_matrix(), dtype=jnp.float32)
        tc_out = pl.pallas_call(
            functools.partial(_tc_body, grid=_TC_BLOCKS, scale=scale),
            grid=(_TC_BLOCKS,),
            in_specs=[
                pl.BlockSpec((kd, _TC_BLOCK), lambda i: (0, i)),
                pl.BlockSpec(wmat.shape, lambda i: (0, 0)),
            ],
            out_specs=pl.BlockSpec((1, 1), lambda i: (0, 0),
                                   memory_space=pltpu.SMEM),
            out_shape=jax.ShapeDtypeStruct((1, 1), jnp.float32),
        )(xt, wmat)
        total = total + tc_out[0, 0]
    return total


# speedup vs baseline: 2.5656x; 2.5656x over previous
import functools

import numpy as np
import jax
import jax.numpy as jnp
from jax import lax
from jax.experimental import pallas as pl
from jax.experimental.pallas import tpu as pltpu
from jax.experimental.pallas import tpu_sc as plsc

_K = 3
_D = 100
_M1 = 0.05
_M2 = 0.25
_REG = 10.0

_SLAB = 128
_NW = 32
_TC_BLOCK = 8192
_TC_BLOCKS = 10


def _hinge(d):
    return jnp.maximum(_M1 - d, 0.0) + jnp.maximum(d - _M2, 0.0)


def _band_matrix():
    w = np.zeros((_K, _K * _D), np.float32)
    for p in range(_K):
        w[p, p * _D:(p + 1) * _D] = 1.0
    return w


def _tc_body(x_ref, w_ref, o_ref, *, grid, scale):
    i = pl.program_id(0)
    x = x_ref[...]
    r = jnp.roll(x, -_D, axis=0)
    z = (x - r) ** 2
    d = jax.lax.dot_general(w_ref[...], z, (((1,), (0,)), ((), ())),
                            preferred_element_type=jnp.float32)
    h = _hinge(d)
    s = jnp.sum(h)

    @pl.when(i == 0)
    def _init():
        o_ref[0, 0] = 0.0

    o_ref[0, 0] += s

    @pl.when(i == grid - 1)
    def _fin():
        o_ref[0, 0] *= scale


def kernel(user_ids, pos_ids, neg_ids, user_emb, item_emb):
    n, kd = user_emb.shape
    xt = user_emb.T
    scale = 2.0 * _REG / (n * _K * _K)
    c0 = _TC_BLOCKS * _TC_BLOCK
    slab0 = c0 // _SLAB
    nslabs = n // _SLAB
    tail = n - nslabs * _SLAB
    mesh = plsc.VectorSubcoreMesh(core_axis_name="c", subcore_axis_name="s")

    @functools.partial(
        pl.kernel,
        mesh=mesh,
        out_type=jax.ShapeDtypeStruct((_NW, 16), jnp.float32),
        scratch_types=[
            pltpu.VMEM((kd, _SLAB), jnp.float32),
            pltpu.VMEM((kd, 32), jnp.float32),
            pltpu.VMEM((16,), jnp.float32),
        ],
    )
    def sck(xt_hbm, out_hbm, buf, buft, accv):
        c = lax.axis_index("c")
        s = lax.axis_index("s")
        wid = s * 2 + c
        accv[...] = jnp.zeros((16,), jnp.float32)
        nj = (nslabs - slab0 + _NW - 1) // _NW

        def lane_groups(b, width):
            for l in range(width // 16):
                def row_body(r, carry):
                    d01, d12, d20 = carry
                    v0 = b[r, pl.ds(l * 16, 16)]
                    v1 = b[r + _D, pl.ds(l * 16, 16)]
                    v2 = b[r + 2 * _D, pl.ds(l * 16, 16)]
                    e0 = v0 - v1
                    e1 = v1 - v2
                    e2 = v2 - v0
                    return (d01 + e0 * e0, d12 + e1 * e1, d20 + e2 * e2)

                z = jnp.zeros((16,), jnp.float32)
                d01, d12, d20 = lax.fori_loop(0, _D, row_body, (z, z, z))
                accv[...] = (accv[...]
                             + _hinge(d01) + _hinge(d12) + _hinge(d20))

        def slab_body(j, _):
            sl = slab0 + wid + j * _NW

            @pl.when(sl < nslabs)
            def _do():
                pltpu.sync_copy(xt_hbm.at[:, pl.ds(sl * _SLAB, _SLAB)], buf)
                lane_groups(buf, _SLAB)
            return ()

        lax.fori_loop(0, nj, slab_body, ())

        if tail:
            @pl.when(wid == _NW - 1)
            def _tail():
                pltpu.sync_copy(xt_hbm.at[:, pl.ds(nslabs * _SLAB, tail)],
                                buft)
                lane_groups(buft, tail)

        accv[...] = accv[...] * scale
        pltpu.sync_copy(accv, out_hbm.at[wid])

    total = jnp.sum(sck(xt))

    if _TC_BLOCKS:
        wmat = jnp.asarray(_band_matrix(), dtype=jnp.float32)
        tc_out = pl.pallas_call(
            functools.partial(_tc_body, grid=_TC_BLOCKS, scale=scale),
            grid=(_TC_BLOCKS,),
            in_specs=[
                pl.BlockSpec((kd, _TC_BLOCK), lambda i: (0, i)),
                pl.BlockSpec(wmat.shape, lambda i: (0, 0)),
            ],
            out_specs=pl.BlockSpec((1, 1), lambda i: (0, 0),
                                   memory_space=pltpu.SMEM),
            out_shape=jax.ShapeDtypeStruct((1, 1), jnp.float32),
        )(xt, wmat)
        total = total + tc_out[0, 0]
    return total


# speedup vs baseline: 3.7614x; 1.4661x over previous
import functools

import numpy as np
import jax
import jax.numpy as jnp
from jax.experimental import pallas as pl
from jax.experimental.pallas import tpu as pltpu

_K = 3
_D = 100
_M1 = 0.05
_M2 = 0.25
_REG = 10.0


def _band_matrix():
    w = np.zeros((_K, _K * _D), np.float32)
    for p in range(_K):
        w[p, p * _D:(p + 1) * _D] = 1.0
    return w


def _body(x1_ref, x2_ref, w_ref, o_ref, *, grid, ncols, block, scale):
    i = pl.program_id(0)
    s = 0.0
    for half, xr in enumerate((x1_ref, x2_ref)):
        x = xr[...]
        r = jnp.roll(x, -_D, axis=0)
        z = (x - r) ** 2
        d = jax.lax.dot_general(w_ref[...], z, (((1,), (0,)), ((), ())),
                                preferred_element_type=jnp.float32)
        h = jnp.maximum(_M1 - d, 0.0) + jnp.maximum(d - _M2, 0.0)
        col = (jax.lax.broadcasted_iota(jnp.int32, h.shape, 1)
               + (2 * i + half) * (block // 2))
        s = s + jnp.sum(jnp.where(col < ncols, h, 0.0))

    @pl.when(i == 0)
    def _init():
        o_ref[0, 0] = 0.0

    o_ref[0, 0] += s

    @pl.when(i == grid - 1)
    def _fin():
        o_ref[0, 0] *= scale


def kernel(user_ids, pos_ids, neg_ids, user_emb, item_emb):
    n, kd = user_emb.shape
    xt = user_emb.T
    block = 11264
    grid = (n + block - 1) // block
    scale = 2.0 * _REG / (n * _K * _K)
    wmat = jnp.asarray(_band_matrix(), dtype=jnp.float32)
    out = pl.pallas_call(
        functools.partial(_body, grid=grid, ncols=n, block=block,
                          scale=scale),
        grid=(grid,),
        in_specs=[
            pl.BlockSpec((kd, block // 2), lambda i: (0, 2 * i)),
            pl.BlockSpec((kd, block // 2), lambda i: (0, 2 * i + 1)),
            pl.BlockSpec(wmat.shape, lambda i: (0, 0)),
        ],
        out_specs=pl.BlockSpec((1, 1), lambda i: (0, 0),
                               memory_space=pltpu.SMEM),
        out_shape=jax.ShapeDtypeStruct((1, 1), jnp.float32),
    )(xt, xt, wmat)
    return out[0, 0]
